# baseline (speedup 1.0000x reference)
import jax
import jax.numpy as jnp
from jax.experimental import pallas as pl
from jax.experimental.pallas import tpu as pltpu

_BH = 200
_CS = 2000


def _fused_body(w_ref, f_hbm, adj_ref, o_ref, s_ref, f_buf, sem):
    t = pl.program_id(0)
    n_rows = f_hbm.shape[0]

    @pl.when(t == 0)
    def _support():
        n_chunks = n_rows // _CS

        def chunk_copy(j):
            return pltpu.make_async_copy(
                f_hbm.at[pl.ds(j * _CS, _CS), :], f_buf.at[j % 2], sem.at[j % 2]
            )

        chunk_copy(0).start()
        for j in range(n_chunks):
            if j + 1 < n_chunks:
                chunk_copy(j + 1).start()
            chunk_copy(j).wait()
            s_ref[j * _CS:(j + 1) * _CS, :] = jnp.dot(
                f_buf[j % 2].astype(jnp.bfloat16),
                w_ref[...],
                preferred_element_type=jnp.float32,
            ).astype(jnp.bfloat16)

    n = f_hbm.shape[0]
    a = adj_ref[...].astype(jnp.bfloat16).reshape(2 * _BH, n)
    acc = jnp.dot(a, s_ref[...], preferred_element_type=jnp.float32)
    o_ref[...] = jnp.maximum(acc, 0.0).reshape(2, _BH, s_ref.shape[1])


def kernel(features, adj, weight):
    n, d_in = features.shape
    d_out = weight.shape[1]
    adj3 = adj.reshape(2, n // 2, n)

    out = pl.pallas_call(
        _fused_body,
        grid=(n // 2 // _BH,),
        in_specs=[
            pl.BlockSpec((d_in, d_out), lambda i: (0, 0)),
            pl.BlockSpec(memory_space=pl.ANY),
            pl.BlockSpec((2, _BH, n), lambda i: (0, i, 0)),
        ],
        out_specs=pl.BlockSpec((2, _BH, d_out), lambda i: (0, i, 0)),
        out_shape=jax.ShapeDtypeStruct((2, n // 2, d_out), jnp.float32),
        scratch_shapes=[
            pltpu.VMEM((n, d_out), jnp.bfloat16),
            pltpu.VMEM((2, _CS, d_in), jnp.float32),
            pltpu.SemaphoreType.DMA((2,)),
        ],
        compiler_params=pltpu.CompilerParams(
            dimension_semantics=("arbitrary",),
        ),
    )(weight.astype(jnp.bfloat16), features, adj3)

    return out.reshape(n, d_out)
